# baseline (speedup 1.0000x reference)
import jax
import jax.numpy as jnp
from jax.experimental import pallas as pl
from jax.experimental.pallas import tpu as pltpu
from jax.experimental.pallas import tpu_sc as plsc

B, T, D, H = 16, 512, 256, 256
G = 4 * H
C = 64
NCH = T // C


def _lstm_kernel(x_ref, perm_ref, slen_ref, wih_ref, whh_ref, bih_ref, bhh_ref,
                 out_ref, hs_ref, cs_ref,
                 xb_ref, ob_ref, xp_ref, zb_ref, h_ref, c_ref,
                 in_sems, out_sems, zsem):
    lens = slen_ref[...]
    maxlen = jnp.max(lens)
    nchunks = (maxlen + (C - 1)) // C

    bias = bih_ref[...] + bhh_ref[...]
    wih = wih_ref[...].astype(jnp.bfloat16)
    whh = whh_ref[...].astype(jnp.bfloat16)

    def in_copy(ci, s, b):
        return pltpu.make_async_copy(
            x_ref.at[perm_ref[b], pl.ds(ci * C, C), :],
            xb_ref.at[s, :, b, :], in_sems.at[s])

    def out_copy(ci, s, b):
        return pltpu.make_async_copy(
            ob_ref.at[s, :, b, :],
            out_ref.at[b, pl.ds(ci * C, C), :], out_sems.at[s])

    @pl.when(nchunks > 0)
    def _prefetch0():
        for b in range(B):
            in_copy(0, 0, b).start()

    zb_ref[...] = jnp.zeros_like(zb_ref)

    def _ztail(ci, carry):
        for b in range(B):
            pltpu.make_async_copy(
                zb_ref, out_ref.at[b, pl.ds(ci * C, C), :], zsem).start()
        return carry

    jax.lax.fori_loop(nchunks, NCH, _ztail, 0)

    h_ref[...] = jnp.zeros_like(h_ref)
    c_ref[...] = jnp.zeros_like(c_ref)

    for ci in range(NCH):
        s = ci % 2

        @pl.when(ci < nchunks)
        def _chunk(ci=ci, s=s):
            if ci + 1 < NCH:
                @pl.when(ci + 1 < nchunks)
                def _prefetch():
                    for b in range(B):
                        in_copy(ci + 1, (ci + 1) % 2, b).start()
            for b in range(B):
                in_copy(ci, s, b).wait()

            xp_ref[...] = (jnp.dot(
                xb_ref[s].reshape(C * B, D).astype(jnp.bfloat16), wih,
                preferred_element_type=jnp.float32) + bias).astype(jnp.bfloat16)

            if ci >= 2:
                for b in range(B):
                    out_copy(ci - 2, s, b).wait()

            t0 = ci * C

            def step(cc, carry2):
                h2, c2 = carry2
                t = t0 + cc
                gates = xp_ref[pl.ds(cc * B, B), :].astype(jnp.float32) + jnp.dot(
                    h2.astype(jnp.bfloat16), whh,
                    preferred_element_type=jnp.float32)
                sig_if = jax.nn.sigmoid(gates[:, :2 * H])
                i_g = sig_if[:, :H]
                f_g = sig_if[:, H:]
                g_g = jnp.tanh(gates[:, 2 * H:3 * H])
                o_g = jax.nn.sigmoid(gates[:, 3 * H:])
                nc = f_g * c2 + i_g * g_g
                nh = o_g * jnp.tanh(nc)
                active = t < lens
                c3 = jnp.where(active, nc, c2)
                h3 = jnp.where(active, nh, h2)
                outv = jnp.where(active, nh, 0.0)
                ob_ref[s, pl.ds(cc, 1), :, :] = outv[None]
                return (h3, c3)

            hf, cf = jax.lax.fori_loop(0, C, step, (h_ref[...], c_ref[...]),
                                       unroll=16)
            h_ref[...] = hf
            c_ref[...] = cf

            for b in range(B):
                out_copy(ci, s, b).start()

    @pl.when(nchunks >= 2)
    def _drain2():
        for b in range(B):
            out_copy(0, 0, b).wait()
            out_copy(0, 1, b).wait()

    @pl.when(nchunks == 1)
    def _drain1():
        for b in range(B):
            out_copy(0, 0, b).wait()

    def _zwait(ci, carry):
        for b in range(B):
            pltpu.make_async_copy(
                zb_ref, out_ref.at[b, pl.ds(0, C), :], zsem).wait()
        return carry

    jax.lax.fori_loop(nchunks, NCH, _zwait, 0)

    hs_ref[...] = h_ref[...]
    cs_ref[...] = c_ref[...]


def _sc_sort(lengths):
    @pl.kernel(
        out_type=[jax.ShapeDtypeStruct((B,), jnp.int32)] * 3,
        mesh=plsc.VectorSubcoreMesh(core_axis_name="c", subcore_axis_name="s"),
        scratch_types=[pltpu.VMEM((B,), jnp.int32) for _ in range(4)]
                      + [pltpu.SemaphoreType.DMA],
        compiler_params=pltpu.CompilerParams(needs_layout_passes=False),
    )
    def sort_kernel(len_ref, perm_ref, restor_ref, slen_ref,
                    lv_ref, pv_ref, rv_ref, sv_ref, sem):
        c = jax.lax.axis_index("c")
        s = jax.lax.axis_index("s")

        @pl.when(jnp.logical_and(c == 0, s == 0))
        def _():
            pltpu.async_copy(len_ref, lv_ref, sem).wait()
            lens = lv_ref[...]
            idx = jax.lax.iota(jnp.int32, B)
            key = lens * B + (B - 1 - idx)
            skey, perm = plsc.sort_key_val(key, idx, descending=True)
            _, restor = plsc.sort_key_val(perm, idx)
            pv_ref[...] = perm
            rv_ref[...] = restor
            sv_ref[...] = skey // B
            pltpu.async_copy(pv_ref, perm_ref, sem).wait()
            pltpu.async_copy(rv_ref, restor_ref, sem).wait()
            pltpu.async_copy(sv_ref, slen_ref, sem).wait()

    return sort_kernel(lengths)


@jax.jit
def kernel(inputs, mask, W_ih, W_hh, b_ih, b_hh):
    mask = mask.astype(jnp.int32)
    lengths = mask.sum(-1)
    permutation, restoration, sorted_lengths = _sc_sort(lengths)

    outputs, hs, cs = pl.pallas_call(
        _lstm_kernel,
        in_specs=[
            pl.BlockSpec(memory_space=pltpu.MemorySpace.HBM),
            pl.BlockSpec(memory_space=pltpu.MemorySpace.SMEM),
            pl.BlockSpec(memory_space=pltpu.MemorySpace.VMEM),
            pl.BlockSpec(memory_space=pltpu.MemorySpace.VMEM),
            pl.BlockSpec(memory_space=pltpu.MemorySpace.VMEM),
            pl.BlockSpec(memory_space=pltpu.MemorySpace.VMEM),
            pl.BlockSpec(memory_space=pltpu.MemorySpace.VMEM),
        ],
        out_specs=[
            pl.BlockSpec(memory_space=pltpu.MemorySpace.HBM),
            pl.BlockSpec(memory_space=pltpu.MemorySpace.VMEM),
            pl.BlockSpec(memory_space=pltpu.MemorySpace.VMEM),
        ],
        out_shape=[
            jax.ShapeDtypeStruct((B, T, H), jnp.float32),
            jax.ShapeDtypeStruct((B, H), jnp.float32),
            jax.ShapeDtypeStruct((B, H), jnp.float32),
        ],
        scratch_shapes=[
            pltpu.VMEM((2, C, B, D), jnp.float32),
            pltpu.VMEM((2, C, B, H), jnp.float32),
            pltpu.VMEM((C * B, G), jnp.bfloat16),
            pltpu.VMEM((C, H), jnp.float32),
            pltpu.VMEM((B, H), jnp.float32),
            pltpu.VMEM((B, H), jnp.float32),
            pltpu.SemaphoreType.DMA((2,)),
            pltpu.SemaphoreType.DMA((2,)),
            pltpu.SemaphoreType.DMA,
        ],
    )(inputs, permutation, sorted_lengths[:, None],
      W_ih.T, W_hh.T, b_ih[None, :], b_hh[None, :])

    return outputs, hs[None], cs[None], restoration


# speedup vs baseline: 1.2518x; 1.2518x over previous
import jax
import jax.numpy as jnp
from jax.experimental import pallas as pl
from jax.experimental.pallas import tpu as pltpu

B, T, D, H = 16, 512, 256, 256
G = 4 * H
C = 64
NCH = T // C


def _lstm_kernel(x_ref, mask_ref, wih_ref, whh_ref, bih_ref, bhh_ref,
                 out_ref, hs_ref, cs_ref, restor_ref,
                 xb_ref, ob_ref, xp_ref, zb_ref, h_ref, c_ref,
                 pv_ref, psm_ref,
                 in_sems, out_sems, zsem, psem):

    lens_col = jnp.sum(mask_ref[...], axis=1, keepdims=True).astype(jnp.float32)
    i0 = jax.lax.broadcasted_iota(jnp.int32, (B, B), 0)
    i1 = jax.lax.broadcasted_iota(jnp.int32, (B, B), 1)
    eye = i0 == i1
    key_col = lens_col * B + (
        B - 1 - jax.lax.broadcasted_iota(jnp.int32, (B, 1), 0)).astype(jnp.float32)
    key_row = jnp.sum(jnp.where(eye, key_col, 0.0), axis=0, keepdims=True)
    restor_row = jnp.sum((key_col > key_row).astype(jnp.int32),
                         axis=0, keepdims=True)
    restor_ref[...] = restor_row
    P = (i0 == restor_row).astype(jnp.float32)
    permv = jnp.sum(P * i1.astype(jnp.float32), axis=1, keepdims=True)
    pv_ref[...] = permv.astype(jnp.int32)
    lens_row = jnp.sum(jnp.where(eye, lens_col, 0.0), axis=0, keepdims=True)
    lens = jnp.sum(P * lens_row, axis=1, keepdims=True).astype(jnp.int32)
    pltpu.make_async_copy(pv_ref, psm_ref, psem).start()

    maxlen = jnp.max(lens)
    nchunks = (maxlen + (C - 1)) // C

    bias = bih_ref[...] + bhh_ref[...]
    wih = wih_ref[...].astype(jnp.bfloat16)
    whh = whh_ref[...].astype(jnp.bfloat16)

    pltpu.make_async_copy(pv_ref, psm_ref, psem).wait()

    def in_copy(ci, s, b):
        return pltpu.make_async_copy(
            x_ref.at[psm_ref[b, 0], pl.ds(ci * C, C), :],
            xb_ref.at[s, :, b, :], in_sems.at[s])

    def out_copy(ci, s, b):
        return pltpu.make_async_copy(
            ob_ref.at[s, :, b, :],
            out_ref.at[b, pl.ds(ci * C, C), :], out_sems.at[s])

    @pl.when(nchunks > 0)
    def _prefetch0():
        for b in range(B):
            in_copy(0, 0, b).start()

    zb_ref[...] = jnp.zeros_like(zb_ref)

    def _ztail(ci, carry):
        for b in range(B):
            pltpu.make_async_copy(
                zb_ref, out_ref.at[b, pl.ds(ci * C, C), :], zsem).start()
        return carry

    jax.lax.fori_loop(nchunks, NCH, _ztail, 0)

    h_ref[...] = jnp.zeros_like(h_ref)
    c_ref[...] = jnp.zeros_like(c_ref)

    for ci in range(NCH):
        s = ci % 2

        @pl.when(ci < nchunks)
        def _chunk(ci=ci, s=s):
            if ci + 1 < NCH:
                @pl.when(ci + 1 < nchunks)
                def _prefetch():
                    for b in range(B):
                        in_copy(ci + 1, (ci + 1) % 2, b).start()
            for b in range(B):
                in_copy(ci, s, b).wait()

            xp_ref[...] = (jnp.dot(
                xb_ref[s].reshape(C * B, D).astype(jnp.bfloat16), wih,
                preferred_element_type=jnp.float32) + bias).astype(jnp.bfloat16)

            if ci >= 2:
                for b in range(B):
                    out_copy(ci - 2, s, b).wait()

            t0 = ci * C

            def step(cc, carry2):
                h2, c2 = carry2
                t = t0 + cc
                gates = xp_ref[pl.ds(cc * B, B), :].astype(jnp.float32) + jnp.dot(
                    h2.astype(jnp.bfloat16), whh,
                    preferred_element_type=jnp.float32)
                sig_if = jax.nn.sigmoid(gates[:, :2 * H])
                i_g = sig_if[:, :H]
                f_g = sig_if[:, H:]
                g_g = jnp.tanh(gates[:, 2 * H:3 * H])
                o_g = jax.nn.sigmoid(gates[:, 3 * H:])
                nc = f_g * c2 + i_g * g_g
                nh = o_g * jnp.tanh(nc)
                active = t < lens
                c3 = jnp.where(active, nc, c2)
                h3 = jnp.where(active, nh, h2)
                outv = jnp.where(active, nh, 0.0)
                ob_ref[s, pl.ds(cc, 1), :, :] = outv[None]
                return (h3, c3)

            hf, cf = jax.lax.fori_loop(0, C, step, (h_ref[...], c_ref[...]),
                                       unroll=16)
            h_ref[...] = hf
            c_ref[...] = cf

            for b in range(B):
                out_copy(ci, s, b).start()

    @pl.when(nchunks >= 2)
    def _drain2():
        for b in range(B):
            out_copy(0, 0, b).wait()
            out_copy(0, 1, b).wait()

    @pl.when(nchunks == 1)
    def _drain1():
        for b in range(B):
            out_copy(0, 0, b).wait()

    def _zwait(ci, carry):
        for b in range(B):
            pltpu.make_async_copy(
                zb_ref, out_ref.at[b, pl.ds(0, C), :], zsem).wait()
        return carry

    jax.lax.fori_loop(nchunks, NCH, _zwait, 0)

    hs_ref[...] = h_ref[...]
    cs_ref[...] = c_ref[...]


@jax.jit
def kernel(inputs, mask, W_ih, W_hh, b_ih, b_hh):
    mask = mask.astype(jnp.int32)

    outputs, hs, cs, restor = pl.pallas_call(
        _lstm_kernel,
        in_specs=[
            pl.BlockSpec(memory_space=pltpu.MemorySpace.HBM),
            pl.BlockSpec(memory_space=pltpu.MemorySpace.VMEM),
            pl.BlockSpec(memory_space=pltpu.MemorySpace.VMEM),
            pl.BlockSpec(memory_space=pltpu.MemorySpace.VMEM),
            pl.BlockSpec(memory_space=pltpu.MemorySpace.VMEM),
            pl.BlockSpec(memory_space=pltpu.MemorySpace.VMEM),
        ],
        out_specs=[
            pl.BlockSpec(memory_space=pltpu.MemorySpace.HBM),
            pl.BlockSpec(memory_space=pltpu.MemorySpace.VMEM),
            pl.BlockSpec(memory_space=pltpu.MemorySpace.VMEM),
            pl.BlockSpec(memory_space=pltpu.MemorySpace.VMEM),
        ],
        out_shape=[
            jax.ShapeDtypeStruct((B, T, H), jnp.float32),
            jax.ShapeDtypeStruct((B, H), jnp.float32),
            jax.ShapeDtypeStruct((B, H), jnp.float32),
            jax.ShapeDtypeStruct((1, B), jnp.int32),
        ],
        scratch_shapes=[
            pltpu.VMEM((2, C, B, D), jnp.float32),
            pltpu.VMEM((2, C, B, H), jnp.float32),
            pltpu.VMEM((C * B, G), jnp.bfloat16),
            pltpu.VMEM((C, H), jnp.float32),
            pltpu.VMEM((B, H), jnp.float32),
            pltpu.VMEM((B, H), jnp.float32),
            pltpu.VMEM((B, 1), jnp.int32),
            pltpu.SMEM((B, 1), jnp.int32),
            pltpu.SemaphoreType.DMA((2,)),
            pltpu.SemaphoreType.DMA((2,)),
            pltpu.SemaphoreType.DMA,
            pltpu.SemaphoreType.DMA,
        ],
    )(inputs, mask, W_ih.T, W_hh.T, b_ih[None, :], b_hh[None, :])

    return outputs, hs[None], cs[None], restor.reshape(B)
